# baseline (speedup 1.0000x reference)
import functools

import jax
import jax.numpy as jnp
from jax import lax
from jax.experimental import pallas as pl
from jax.experimental.pallas import tpu as pltpu
from jax.experimental.pallas import tpu_sc as plsc

NC, NS, L = 2, 16, 16
NW = NC * NS

BATCH = 4
SEQ = 8192
EMB = 1024

ROWS_PER_W = SEQ // NW
CHUNK_ROWS = 8
CHUNK = CHUNK_ROWS * EMB
N_CHUNKS = ROWS_PER_W // CHUNK_ROWS


def _sc_body(x_hbm, t_hbm, out_hbm, xbuf, tbuf):
    wid = lax.axis_index("s") * NC + lax.axis_index("c")
    base = wid * ROWS_PER_W * EMB

    def chunk_body(c, _):
        t_off = base + c * CHUNK
        pltpu.sync_copy(t_hbm.at[pl.ds(t_off, CHUNK)], tbuf)

        def batch_body(b, _):
            x_off = b * (SEQ * EMB) + t_off
            pltpu.sync_copy(x_hbm.at[pl.ds(x_off, CHUNK)], xbuf)

            def add_body(j, _):
                o = j * L
                xbuf[pl.ds(o, L)] = xbuf[pl.ds(o, L)] + tbuf[pl.ds(o, L)]
                return 0

            lax.fori_loop(0, CHUNK // L, add_body, 0, unroll=8)
            pltpu.sync_copy(xbuf, out_hbm.at[pl.ds(x_off, CHUNK)])
            return 0

        lax.fori_loop(0, BATCH, batch_body, 0)
        return 0

    lax.fori_loop(0, N_CHUNKS, chunk_body, 0)


@functools.cache
def _make_sc_kernel():
    return pl.kernel(
        _sc_body,
        out_type=jax.ShapeDtypeStruct((BATCH * SEQ * EMB,), jnp.float32),
        mesh=plsc.VectorSubcoreMesh(core_axis_name="c", subcore_axis_name="s"),
        scratch_types=[
            pltpu.VMEM((CHUNK,), jnp.float32),
            pltpu.VMEM((CHUNK,), jnp.float32),
        ],
    )


@jax.jit
def kernel(x, pos_table):
    out = _make_sc_kernel()(x.reshape(-1), pos_table.reshape(-1))
    return out.reshape(x.shape)
